# Initial kernel scaffold; baseline (speedup 1.0000x reference)
import jax
import jax.numpy as jnp
from jax.experimental import pallas as pl


def kernel(x, edge_index, batch, W1a, b1a, W1b, b1b, W2a, b2a, W2b, b2b, Wfc, bfc):
    raise NotImplementedError("write your pallas kernel here")


# speedup vs baseline: 3.5291x; 3.5291x over previous
import functools

import jax
import jax.numpy as jnp
from jax import lax
from jax.experimental import pallas as pl
from jax.experimental.pallas import tpu as pltpu
from jax.experimental.pallas import tpu_sc as plsc

N_NODES = 10000
N_EDGES = 320000
D_FEAT = 128
HIDDEN = 64
N_CLASSES = 2
N_GRAPHS = 16

NC = 2
NS = 16
NW = NC * NS

NPAD = 10240
CH = 128
EPW = 10240
EPAD = EPW * NW
CHUNKS = EPW // CH
RPW = NPAD // NS

_SC_MESH = plsc.VectorSubcoreMesh(
    core_axis_name="c", subcore_axis_name="s", num_cores=NC, num_subcores=NS)


def _agg_body(y_ref, src_ref, dst_ref, zero_ref, out_ref,
              src_v, dst_v, rows_v, acc):
  c = lax.axis_index("c")
  s = lax.axis_index("s")
  w = c * NS + s
  rbase = s * RPW
  pltpu.sync_copy(zero_ref.at[pl.ds(rbase, RPW)], acc.at[pl.ds(rbase, RPW)])
  plsc.subcore_barrier()

  ebase = w * EPW

  def chunk(i, carry):
    off = ebase + i * CH
    pltpu.sync_copy(src_ref.at[pl.ds(off, CH)], src_v)
    pltpu.sync_copy(dst_ref.at[pl.ds(off, CH)], dst_v)
    pltpu.sync_copy(y_ref.at[src_v], rows_v)
    pltpu.sync_copy(rows_v, acc.at[dst_v], add=True)
    return carry

  lax.fori_loop(0, CHUNKS, chunk, 0)
  plsc.subcore_barrier()
  pltpu.sync_copy(acc.at[pl.ds(rbase, RPW)], out_ref.at[c, pl.ds(rbase, RPW)])


_sc_aggregate = pl.kernel(
    _agg_body,
    out_type=jax.ShapeDtypeStruct((NC, NPAD, HIDDEN), jnp.float32),
    mesh=_SC_MESH,
    scratch_types=[
        pltpu.VMEM((CH,), jnp.int32),
        pltpu.VMEM((CH,), jnp.int32),
        pltpu.VMEM((CH, HIDDEN), jnp.float32),
        pltpu.VMEM_SHARED((NPAD, HIDDEN), jnp.float32),
    ],
    compiler_params=pltpu.CompilerParams(use_tc_tiling_on_sc=False),
)


BLK = 1024
NB1 = NPAD // BLK

BLK3 = 512
NB3 = NPAD // BLK3


def _proj_body(x_ref, w_ref, o_ref):
  o_ref[...] = lax.dot_general(
      x_ref[...], w_ref[...], (((1,), (1,)), ((), ())),
      preferred_element_type=jnp.float32)


_proj = pl.pallas_call(
    _proj_body,
    grid=(NB1,),
    in_specs=[
        pl.BlockSpec((BLK, D_FEAT), lambda i: (i, 0)),
        pl.BlockSpec((HIDDEN, D_FEAT), lambda i: (0, 0)),
    ],
    out_specs=pl.BlockSpec((BLK, HIDDEN), lambda i: (i, 0)),
    out_shape=jax.ShapeDtypeStruct((NPAD, HIDDEN), jnp.float32),
)


def _stage2_body(y_ref, a0_ref, a1_ref, ba_ref, wb_ref, bb_ref, wn_ref, o_ref):
  u = jnp.maximum(y_ref[...] + a0_ref[...] + a1_ref[...] + ba_ref[...], 0.0)
  h = jnp.maximum(
      lax.dot_general(u, wb_ref[...], (((1,), (1,)), ((), ())),
                      preferred_element_type=jnp.float32) + bb_ref[...], 0.0)
  o_ref[...] = lax.dot_general(
      h, wn_ref[...], (((1,), (1,)), ((), ())),
      preferred_element_type=jnp.float32)


_stage2 = pl.pallas_call(
    _stage2_body,
    grid=(NB1,),
    in_specs=[
        pl.BlockSpec((BLK, HIDDEN), lambda i: (i, 0)),
        pl.BlockSpec((BLK, HIDDEN), lambda i: (i, 0)),
        pl.BlockSpec((BLK, HIDDEN), lambda i: (i, 0)),
        pl.BlockSpec((1, HIDDEN), lambda i: (0, 0)),
        pl.BlockSpec((HIDDEN, HIDDEN), lambda i: (0, 0)),
        pl.BlockSpec((1, HIDDEN), lambda i: (0, 0)),
        pl.BlockSpec((HIDDEN, HIDDEN), lambda i: (0, 0)),
    ],
    out_specs=pl.BlockSpec((BLK, HIDDEN), lambda i: (i, 0)),
    out_shape=jax.ShapeDtypeStruct((NPAD, HIDDEN), jnp.float32),
)


def _stage3_body(y_ref, a0_ref, a1_ref, ba_ref, wb_ref, bb_ref, batch_ref,
                 wfc_ref, bfc_ref, o_ref, sums_ref, cnts_ref):
  i = pl.program_id(0)

  @pl.when(i == 0)
  def _():
    sums_ref[...] = jnp.zeros_like(sums_ref)
    cnts_ref[...] = jnp.zeros_like(cnts_ref)

  u = jnp.maximum(y_ref[...] + a0_ref[...] + a1_ref[...] + ba_ref[...], 0.0)
  h = jnp.maximum(
      lax.dot_general(u, wb_ref[...], (((1,), (1,)), ((), ())),
                      preferred_element_type=jnp.float32) + bb_ref[...], 0.0)
  bids = batch_ref[0]
  gids = lax.broadcasted_iota(jnp.int32, (N_GRAPHS, BLK3), 0)
  m = (bids == gids).astype(jnp.float32)
  sums_ref[...] += lax.dot_general(
      m, h, (((1,), (0,)), ((), ())), preferred_element_type=jnp.float32)
  cnts_ref[...] += jnp.broadcast_to(
      jnp.sum(m, axis=1, keepdims=True), (N_GRAPHS, HIDDEN))

  @pl.when(i == NB3 - 1)
  def _():
    pooled = sums_ref[...] / jnp.maximum(cnts_ref[...], 1.0)
    o_ref[...] = lax.dot_general(
        pooled, wfc_ref[...], (((1,), (1,)), ((), ())),
        preferred_element_type=jnp.float32) + bfc_ref[...]


_stage3 = pl.pallas_call(
    _stage3_body,
    grid=(NB3,),
    in_specs=[
        pl.BlockSpec((BLK3, HIDDEN), lambda i: (i, 0)),
        pl.BlockSpec((BLK3, HIDDEN), lambda i: (i, 0)),
        pl.BlockSpec((BLK3, HIDDEN), lambda i: (i, 0)),
        pl.BlockSpec((1, HIDDEN), lambda i: (0, 0)),
        pl.BlockSpec((HIDDEN, HIDDEN), lambda i: (0, 0)),
        pl.BlockSpec((1, HIDDEN), lambda i: (0, 0)),
        pl.BlockSpec((1, 1, BLK3), lambda i: (i, 0, 0)),
        pl.BlockSpec((N_CLASSES, HIDDEN), lambda i: (0, 0)),
        pl.BlockSpec((1, N_CLASSES), lambda i: (0, 0)),
    ],
    out_specs=pl.BlockSpec((N_GRAPHS, N_CLASSES), lambda i: (0, 0)),
    out_shape=jax.ShapeDtypeStruct((N_GRAPHS, N_CLASSES), jnp.float32),
    scratch_shapes=[
        pltpu.VMEM((N_GRAPHS, HIDDEN), jnp.float32),
        pltpu.VMEM((N_GRAPHS, HIDDEN), jnp.float32),
    ],
)


@jax.jit
def kernel(x, edge_index, batch, W1a, b1a, W1b, b1b, W2a, b2a, W2b, b2b,
           Wfc, bfc):
  ei = edge_index.astype(jnp.int32)
  epad = EPAD - N_EDGES
  src = jnp.pad(ei[0], (0, epad), constant_values=N_NODES)
  dst = jnp.pad(ei[1], (0, epad), constant_values=N_NODES)
  xpad = jnp.pad(x, ((0, NPAD - N_NODES), (0, 0)))
  bpad = jnp.pad(batch.astype(jnp.int32), (0, NPAD - N_NODES),
                 constant_values=N_GRAPHS).reshape(NB3, 1, BLK3)
  zeros = jnp.zeros((NPAD, HIDDEN), jnp.float32)

  y1 = _proj(xpad, W1a)
  a1 = _sc_aggregate(y1, src, dst, zeros)
  y2 = _stage2(y1, a1[0], a1[1], b1a.reshape(1, HIDDEN), W1b,
               b1b.reshape(1, HIDDEN), W2a)
  a2 = _sc_aggregate(y2, src, dst, zeros)
  out = _stage3(y2, a2[0], a2[1], b2a.reshape(1, HIDDEN), W2b,
                b2b.reshape(1, HIDDEN), bpad, Wfc,
                bfc.reshape(1, N_CLASSES))
  return out


# speedup vs baseline: 4.6312x; 1.3123x over previous
import functools

import jax
import jax.numpy as jnp
from jax import lax
from jax.experimental import pallas as pl
from jax.experimental.pallas import tpu as pltpu
from jax.experimental.pallas import tpu_sc as plsc

N_NODES = 10000
N_EDGES = 320000
D_FEAT = 128
HIDDEN = 64
N_CLASSES = 2
N_GRAPHS = 16

NC = 2
NS = 16
NW = NC * NS

NPAD = 10240
CH = 128
EPW = 10240
EPAD = EPW * NW
CHUNKS = EPW // CH
RPW = NPAD // NS

_SC_MESH = plsc.VectorSubcoreMesh(
    core_axis_name="c", subcore_axis_name="s", num_cores=NC, num_subcores=NS)


NBUF = 4
GROUPS = CHUNKS // NBUF


def _agg_body(y_ref, src_ref, dst_ref, zero_ref, out_ref,
              sidx, didx, rows0, rows1, rows2, rows3,
              g0, g1, g2, g3, s0, s1, s2, s3, acc):
  rows = (rows0, rows1, rows2, rows3)
  gsem = (g0, g1, g2, g3)
  ssem = (s0, s1, s2, s3)
  c = lax.axis_index("c")
  s = lax.axis_index("s")
  w = c * NS + s
  rbase = s * RPW
  pltpu.sync_copy(src_ref.at[w], sidx)
  pltpu.sync_copy(dst_ref.at[w], didx)
  pltpu.sync_copy(zero_ref.at[pl.ds(rbase, RPW)], acc.at[pl.ds(rbase, RPW)])
  plsc.subcore_barrier()

  for b in range(NBUF):
    pltpu.async_copy(y_ref.at[sidx.at[b]], rows[b], gsem[b])

  def group(j, carry):
    base = j * NBUF
    for b in range(NBUF):
      pltpu.make_async_copy(y_ref.at[sidx.at[base + b]], rows[b],
                            gsem[b]).wait()
      pltpu.async_copy(rows[b], acc.at[didx.at[base + b]], ssem[b], add=True)
    for b in range(NBUF):
      nxt = jnp.minimum(base + NBUF + b, CHUNKS - 1)
      pltpu.make_async_copy(rows[b], acc.at[didx.at[nxt]], ssem[b]).wait()

      @pl.when(j < GROUPS - 1)
      def _():
        pltpu.async_copy(y_ref.at[sidx.at[nxt]], rows[b], gsem[b])
    return carry

  lax.fori_loop(0, GROUPS, group, 0)
  plsc.subcore_barrier()
  pltpu.sync_copy(acc.at[pl.ds(rbase, RPW)], out_ref.at[c, pl.ds(rbase, RPW)])


_sc_aggregate = pl.kernel(
    _agg_body,
    out_type=jax.ShapeDtypeStruct((NC, NPAD, HIDDEN), jnp.float32),
    mesh=_SC_MESH,
    scratch_types=[
        pltpu.VMEM((CHUNKS, CH), jnp.int32),
        pltpu.VMEM((CHUNKS, CH), jnp.int32),
    ] + [pltpu.VMEM((CH, HIDDEN), jnp.float32) for _ in range(NBUF)]
    + [pltpu.SemaphoreType.DMA for _ in range(2 * NBUF)]
    + [pltpu.VMEM_SHARED((NPAD, HIDDEN), jnp.float32)],
    compiler_params=pltpu.CompilerParams(use_tc_tiling_on_sc=False),
)


BLK = 1024
NB1 = NPAD // BLK

BLK3 = 512
NB3 = NPAD // BLK3


def _proj_body(x_ref, w_ref, o_ref):
  o_ref[...] = lax.dot_general(
      x_ref[...], w_ref[...], (((1,), (1,)), ((), ())),
      preferred_element_type=jnp.float32)


_proj = pl.pallas_call(
    _proj_body,
    grid=(NB1,),
    in_specs=[
        pl.BlockSpec((BLK, D_FEAT), lambda i: (i, 0)),
        pl.BlockSpec((HIDDEN, D_FEAT), lambda i: (0, 0)),
    ],
    out_specs=pl.BlockSpec((BLK, HIDDEN), lambda i: (i, 0)),
    out_shape=jax.ShapeDtypeStruct((NPAD, HIDDEN), jnp.float32),
)


def _stage2_body(y_ref, a0_ref, a1_ref, ba_ref, wb_ref, bb_ref, wn_ref, o_ref):
  u = jnp.maximum(y_ref[...] + a0_ref[...] + a1_ref[...] + ba_ref[...], 0.0)
  h = jnp.maximum(
      lax.dot_general(u, wb_ref[...], (((1,), (1,)), ((), ())),
                      preferred_element_type=jnp.float32) + bb_ref[...], 0.0)
  o_ref[...] = lax.dot_general(
      h, wn_ref[...], (((1,), (1,)), ((), ())),
      preferred_element_type=jnp.float32)


_stage2 = pl.pallas_call(
    _stage2_body,
    grid=(NB1,),
    in_specs=[
        pl.BlockSpec((BLK, HIDDEN), lambda i: (i, 0)),
        pl.BlockSpec((BLK, HIDDEN), lambda i: (i, 0)),
        pl.BlockSpec((BLK, HIDDEN), lambda i: (i, 0)),
        pl.BlockSpec((1, HIDDEN), lambda i: (0, 0)),
        pl.BlockSpec((HIDDEN, HIDDEN), lambda i: (0, 0)),
        pl.BlockSpec((1, HIDDEN), lambda i: (0, 0)),
        pl.BlockSpec((HIDDEN, HIDDEN), lambda i: (0, 0)),
    ],
    out_specs=pl.BlockSpec((BLK, HIDDEN), lambda i: (i, 0)),
    out_shape=jax.ShapeDtypeStruct((NPAD, HIDDEN), jnp.float32),
)


def _stage3_body(y_ref, a0_ref, a1_ref, ba_ref, wb_ref, bb_ref, batch_ref,
                 wfc_ref, bfc_ref, o_ref, sums_ref, cnts_ref):
  i = pl.program_id(0)

  @pl.when(i == 0)
  def _():
    sums_ref[...] = jnp.zeros_like(sums_ref)
    cnts_ref[...] = jnp.zeros_like(cnts_ref)

  u = jnp.maximum(y_ref[...] + a0_ref[...] + a1_ref[...] + ba_ref[...], 0.0)
  h = jnp.maximum(
      lax.dot_general(u, wb_ref[...], (((1,), (1,)), ((), ())),
                      preferred_element_type=jnp.float32) + bb_ref[...], 0.0)
  bids = batch_ref[0]
  gids = lax.broadcasted_iota(jnp.int32, (N_GRAPHS, BLK3), 0)
  m = (bids == gids).astype(jnp.float32)
  sums_ref[...] += lax.dot_general(
      m, h, (((1,), (0,)), ((), ())), preferred_element_type=jnp.float32)
  cnts_ref[...] += jnp.broadcast_to(
      jnp.sum(m, axis=1, keepdims=True), (N_GRAPHS, HIDDEN))

  @pl.when(i == NB3 - 1)
  def _():
    pooled = sums_ref[...] / jnp.maximum(cnts_ref[...], 1.0)
    o_ref[...] = lax.dot_general(
        pooled, wfc_ref[...], (((1,), (1,)), ((), ())),
        preferred_element_type=jnp.float32) + bfc_ref[...]


_stage3 = pl.pallas_call(
    _stage3_body,
    grid=(NB3,),
    in_specs=[
        pl.BlockSpec((BLK3, HIDDEN), lambda i: (i, 0)),
        pl.BlockSpec((BLK3, HIDDEN), lambda i: (i, 0)),
        pl.BlockSpec((BLK3, HIDDEN), lambda i: (i, 0)),
        pl.BlockSpec((1, HIDDEN), lambda i: (0, 0)),
        pl.BlockSpec((HIDDEN, HIDDEN), lambda i: (0, 0)),
        pl.BlockSpec((1, HIDDEN), lambda i: (0, 0)),
        pl.BlockSpec((1, 1, BLK3), lambda i: (i, 0, 0)),
        pl.BlockSpec((N_CLASSES, HIDDEN), lambda i: (0, 0)),
        pl.BlockSpec((1, N_CLASSES), lambda i: (0, 0)),
    ],
    out_specs=pl.BlockSpec((N_GRAPHS, N_CLASSES), lambda i: (0, 0)),
    out_shape=jax.ShapeDtypeStruct((N_GRAPHS, N_CLASSES), jnp.float32),
    scratch_shapes=[
        pltpu.VMEM((N_GRAPHS, HIDDEN), jnp.float32),
        pltpu.VMEM((N_GRAPHS, HIDDEN), jnp.float32),
    ],
)


@jax.jit
def kernel(x, edge_index, batch, W1a, b1a, W1b, b1b, W2a, b2a, W2b, b2b,
           Wfc, bfc):
  ei = edge_index.astype(jnp.int32)
  epad = EPAD - N_EDGES
  src = jnp.pad(ei[0], (0, epad), constant_values=N_NODES).reshape(
      NW, CHUNKS, CH)
  dst = jnp.pad(ei[1], (0, epad), constant_values=N_NODES).reshape(
      NW, CHUNKS, CH)
  xpad = jnp.pad(x, ((0, NPAD - N_NODES), (0, 0)))
  bpad = jnp.pad(batch.astype(jnp.int32), (0, NPAD - N_NODES),
                 constant_values=N_GRAPHS).reshape(NB3, 1, BLK3)
  zeros = jnp.zeros((NPAD, HIDDEN), jnp.float32)

  y1 = _proj(xpad, W1a)
  a1 = _sc_aggregate(y1, src, dst, zeros)
  y2 = _stage2(y1, a1[0], a1[1], b1a.reshape(1, HIDDEN), W1b,
               b1b.reshape(1, HIDDEN), W2a)
  a2 = _sc_aggregate(y2, src, dst, zeros)
  out = _stage3(y2, a2[0], a2[1], b2a.reshape(1, HIDDEN), W2b,
                b2b.reshape(1, HIDDEN), bpad, Wfc,
                bfc.reshape(1, N_CLASSES))
  return out


# speedup vs baseline: 12.9739x; 2.8014x over previous
import functools

import jax
import jax.numpy as jnp
from jax import lax
from jax.experimental import pallas as pl
from jax.experimental.pallas import tpu as pltpu
from jax.experimental.pallas import tpu_sc as plsc

N_NODES = 10000
N_EDGES = 320000
D_FEAT = 128
HIDDEN = 64
N_CLASSES = 2
N_GRAPHS = 16

NC = 2
NS = 16
NW = NC * NS

NPAD = 10240
CH = 128
EPW = 10240
EPAD = EPW * NW
CHUNKS = EPW // CH
RPW = NPAD // NS

_SC_MESH = plsc.VectorSubcoreMesh(
    core_axis_name="c", subcore_axis_name="s", num_cores=NC, num_subcores=NS)


NBUF = 4
GROUPS = CHUNKS // NBUF


def _agg_body(y_ref, src_ref, dst_ref, zero_ref, out_ref,
              sidx, didx, rows0, rows1, rows2, rows3,
              g0, g1, g2, g3, s0, s1, s2, s3, acc):
  rows = (rows0, rows1, rows2, rows3)
  gsem = (g0, g1, g2, g3)
  ssem = (s0, s1, s2, s3)
  c = lax.axis_index("c")
  s = lax.axis_index("s")
  w = c * NS + s
  rbase = s * RPW
  pltpu.sync_copy(src_ref.at[w], sidx)
  pltpu.sync_copy(dst_ref.at[w], didx)
  pltpu.sync_copy(zero_ref.at[pl.ds(rbase, RPW)], acc.at[pl.ds(rbase, RPW)])
  plsc.subcore_barrier()

  for b in range(NBUF):
    pltpu.async_copy(y_ref.at[sidx.at[b]], rows[b], gsem[b])

  def group(j, carry):
    base = j * NBUF
    for b in range(NBUF):
      pltpu.make_async_copy(y_ref.at[sidx.at[base + b]], rows[b],
                            gsem[b]).wait()
      pltpu.async_copy(rows[b], acc.at[didx.at[base + b]], ssem[b], add=True)
    for b in range(NBUF):
      nxt = jnp.minimum(base + NBUF + b, CHUNKS - 1)
      pltpu.make_async_copy(rows[b], acc.at[didx.at[nxt]], ssem[b]).wait()

      @pl.when(j < GROUPS - 1)
      def _():
        pltpu.async_copy(y_ref.at[sidx.at[nxt]], rows[b], gsem[b])
    return carry

  lax.fori_loop(0, GROUPS, group, 0)
  plsc.subcore_barrier()
  pltpu.sync_copy(acc.at[pl.ds(rbase, RPW)], out_ref.at[c, pl.ds(rbase, RPW)])


_sc_aggregate = pl.kernel(
    _agg_body,
    out_type=jax.ShapeDtypeStruct((NC, NPAD, HIDDEN), jnp.float32),
    mesh=_SC_MESH,
    scratch_types=[
        pltpu.VMEM((CHUNKS, CH), jnp.int32),
        pltpu.VMEM((CHUNKS, CH), jnp.int32),
    ] + [pltpu.VMEM((CH, HIDDEN), jnp.float32) for _ in range(NBUF)]
    + [pltpu.SemaphoreType.DMA for _ in range(2 * NBUF)]
    + [pltpu.VMEM_SHARED((NPAD, HIDDEN), jnp.float32)],
    compiler_params=pltpu.CompilerParams(use_tc_tiling_on_sc=False),
)


BLK = 1024
NB1 = NPAD // BLK

BLK3 = 512
NB3 = NPAD // BLK3


def _proj_body(x_ref, w_ref, o_ref):
  o_ref[...] = lax.dot_general(
      x_ref[...], w_ref[...], (((1,), (1,)), ((), ())),
      preferred_element_type=jnp.float32)


_proj = pl.pallas_call(
    _proj_body,
    grid=(NB1,),
    in_specs=[
        pl.BlockSpec((BLK, D_FEAT), lambda i: (i, 0)),
        pl.BlockSpec((HIDDEN, D_FEAT), lambda i: (0, 0)),
    ],
    out_specs=pl.BlockSpec((BLK, HIDDEN), lambda i: (i, 0)),
    out_shape=jax.ShapeDtypeStruct((NPAD, HIDDEN), jnp.float32),
)


def _stage2_body(y_ref, a0_ref, a1_ref, ba_ref, wb_ref, bb_ref, wn_ref, o_ref):
  u = jnp.maximum(y_ref[...] + a0_ref[...] + a1_ref[...] + ba_ref[...], 0.0)
  h = jnp.maximum(
      lax.dot_general(u, wb_ref[...], (((1,), (1,)), ((), ())),
                      preferred_element_type=jnp.float32) + bb_ref[...], 0.0)
  o_ref[...] = lax.dot_general(
      h, wn_ref[...], (((1,), (1,)), ((), ())),
      preferred_element_type=jnp.float32)


_stage2 = pl.pallas_call(
    _stage2_body,
    grid=(NB1,),
    in_specs=[
        pl.BlockSpec((BLK, HIDDEN), lambda i: (i, 0)),
        pl.BlockSpec((BLK, HIDDEN), lambda i: (i, 0)),
        pl.BlockSpec((BLK, HIDDEN), lambda i: (i, 0)),
        pl.BlockSpec((1, HIDDEN), lambda i: (0, 0)),
        pl.BlockSpec((HIDDEN, HIDDEN), lambda i: (0, 0)),
        pl.BlockSpec((1, HIDDEN), lambda i: (0, 0)),
        pl.BlockSpec((HIDDEN, HIDDEN), lambda i: (0, 0)),
    ],
    out_specs=pl.BlockSpec((BLK, HIDDEN), lambda i: (i, 0)),
    out_shape=jax.ShapeDtypeStruct((NPAD, HIDDEN), jnp.float32),
)


def _stage3_body(y_ref, a0_ref, a1_ref, ba_ref, wb_ref, bb_ref, batch_ref,
                 wfc_ref, bfc_ref, o_ref, sums_ref, cnts_ref):
  i = pl.program_id(0)

  @pl.when(i == 0)
  def _():
    sums_ref[...] = jnp.zeros_like(sums_ref)
    cnts_ref[...] = jnp.zeros_like(cnts_ref)

  u = jnp.maximum(y_ref[...] + a0_ref[...] + a1_ref[...] + ba_ref[...], 0.0)
  h = jnp.maximum(
      lax.dot_general(u, wb_ref[...], (((1,), (1,)), ((), ())),
                      preferred_element_type=jnp.float32) + bb_ref[...], 0.0)
  bids = batch_ref[0]
  gids = lax.broadcasted_iota(jnp.int32, (N_GRAPHS, BLK3), 0)
  m = (bids == gids).astype(jnp.float32)
  sums_ref[...] += lax.dot_general(
      m, h, (((1,), (0,)), ((), ())), preferred_element_type=jnp.float32)
  cnts_ref[...] += jnp.broadcast_to(
      jnp.sum(m, axis=1, keepdims=True), (N_GRAPHS, HIDDEN))

  @pl.when(i == NB3 - 1)
  def _():
    pooled = sums_ref[...] / jnp.maximum(cnts_ref[...], 1.0)
    o_ref[...] = lax.dot_general(
        pooled, wfc_ref[...], (((1,), (1,)), ((), ())),
        preferred_element_type=jnp.float32) + bfc_ref[...]


_stage3 = pl.pallas_call(
    _stage3_body,
    grid=(NB3,),
    in_specs=[
        pl.BlockSpec((BLK3, HIDDEN), lambda i: (i, 0)),
        pl.BlockSpec((BLK3, HIDDEN), lambda i: (i, 0)),
        pl.BlockSpec((BLK3, HIDDEN), lambda i: (i, 0)),
        pl.BlockSpec((1, HIDDEN), lambda i: (0, 0)),
        pl.BlockSpec((HIDDEN, HIDDEN), lambda i: (0, 0)),
        pl.BlockSpec((1, HIDDEN), lambda i: (0, 0)),
        pl.BlockSpec((1, 1, BLK3), lambda i: (i, 0, 0)),
        pl.BlockSpec((N_CLASSES, HIDDEN), lambda i: (0, 0)),
        pl.BlockSpec((1, N_CLASSES), lambda i: (0, 0)),
    ],
    out_specs=pl.BlockSpec((N_GRAPHS, N_CLASSES), lambda i: (0, 0)),
    out_shape=jax.ShapeDtypeStruct((N_GRAPHS, N_CLASSES), jnp.float32),
    scratch_shapes=[
        pltpu.VMEM((N_GRAPHS, HIDDEN), jnp.float32),
        pltpu.VMEM((N_GRAPHS, HIDDEN), jnp.float32),
    ],
)


@jax.jit
def kernel(x, edge_index, batch, W1a, b1a, W1b, b1b, W2a, b2a, W2b, b2b,
           Wfc, bfc):
  ei = edge_index.astype(jnp.int32)
  epad = EPAD - N_EDGES
  fill = N_NODES + (jnp.arange(epad, dtype=jnp.int32) % (NPAD - N_NODES))
  src = jnp.concatenate([ei[0], fill]).reshape(NW, CHUNKS, CH)
  dst = jnp.concatenate([ei[1], fill]).reshape(NW, CHUNKS, CH)
  xpad = jnp.pad(x, ((0, NPAD - N_NODES), (0, 0)))
  bpad = jnp.pad(batch.astype(jnp.int32), (0, NPAD - N_NODES),
                 constant_values=N_GRAPHS).reshape(NB3, 1, BLK3)
  zeros = jnp.zeros((NPAD, HIDDEN), jnp.float32)

  y1 = _proj(xpad, W1a)
  a1 = _sc_aggregate(y1, src, dst, zeros)
  y2 = _stage2(y1, a1[0], a1[1], b1a.reshape(1, HIDDEN), W1b,
               b1b.reshape(1, HIDDEN), W2a)
  a2 = _sc_aggregate(y2, src, dst, zeros)
  out = _stage3(y2, a2[0], a2[1], b2a.reshape(1, HIDDEN), W2b,
                b2b.reshape(1, HIDDEN), bpad, Wfc,
                bfc.reshape(1, N_CLASSES))
  return out


# speedup vs baseline: 13.4174x; 1.0342x over previous
import functools

import jax
import jax.numpy as jnp
from jax import lax
from jax.experimental import pallas as pl
from jax.experimental.pallas import tpu as pltpu
from jax.experimental.pallas import tpu_sc as plsc

N_NODES = 10000
N_EDGES = 320000
D_FEAT = 128
HIDDEN = 64
N_CLASSES = 2
N_GRAPHS = 16

NC = 2
NS = 16
NW = NC * NS

NPAD = 10240
CH = 128
EPW = 10240
EPAD = EPW * NW
CHUNKS = EPW // CH
RPW = NPAD // NS

_SC_MESH = plsc.VectorSubcoreMesh(
    core_axis_name="c", subcore_axis_name="s", num_cores=NC, num_subcores=NS)


NBUF = 8
GROUPS = CHUNKS // NBUF


def _agg_body(y_ref, src_ref, dst_ref, zero_ref, out_ref, *scratch):
  sidx, didx = scratch[0], scratch[1]
  rows = scratch[2:2 + NBUF]
  gsem = scratch[2 + NBUF:2 + 2 * NBUF]
  ssem = scratch[2 + 2 * NBUF:2 + 3 * NBUF]
  acc = scratch[2 + 3 * NBUF]
  c = lax.axis_index("c")
  s = lax.axis_index("s")
  w = c * NS + s
  rbase = s * RPW
  pltpu.sync_copy(src_ref.at[w], sidx)
  pltpu.sync_copy(dst_ref.at[w], didx)
  pltpu.sync_copy(zero_ref.at[pl.ds(rbase, RPW)], acc.at[pl.ds(rbase, RPW)])
  plsc.subcore_barrier()

  for b in range(NBUF):
    pltpu.async_copy(y_ref.at[sidx.at[b]], rows[b], gsem[b])

  def group(j, carry):
    base = j * NBUF
    for b in range(NBUF):
      pltpu.make_async_copy(y_ref.at[sidx.at[base + b]], rows[b],
                            gsem[b]).wait()
      pltpu.async_copy(rows[b], acc.at[didx.at[base + b]], ssem[b], add=True)
    for b in range(NBUF):
      nxt = jnp.minimum(base + NBUF + b, CHUNKS - 1)
      pltpu.make_async_copy(rows[b], acc.at[didx.at[nxt]], ssem[b]).wait()

      @pl.when(j < GROUPS - 1)
      def _():
        pltpu.async_copy(y_ref.at[sidx.at[nxt]], rows[b], gsem[b])
    return carry

  lax.fori_loop(0, GROUPS, group, 0)
  plsc.subcore_barrier()
  pltpu.sync_copy(acc.at[pl.ds(rbase, RPW)], out_ref.at[c, pl.ds(rbase, RPW)])


_sc_aggregate = pl.kernel(
    _agg_body,
    out_type=jax.ShapeDtypeStruct((NC, NPAD, HIDDEN), jnp.float32),
    mesh=_SC_MESH,
    scratch_types=[
        pltpu.VMEM((CHUNKS, CH), jnp.int32),
        pltpu.VMEM((CHUNKS, CH), jnp.int32),
    ] + [pltpu.VMEM((CH, HIDDEN), jnp.float32) for _ in range(NBUF)]
    + [pltpu.SemaphoreType.DMA for _ in range(2 * NBUF)]
    + [pltpu.VMEM_SHARED((NPAD, HIDDEN), jnp.float32)],
    compiler_params=pltpu.CompilerParams(use_tc_tiling_on_sc=False),
)


BLK = 1024
NB1 = NPAD // BLK

BLK3 = 512
NB3 = NPAD // BLK3


def _proj_body(x_ref, w_ref, o_ref):
  o_ref[...] = lax.dot_general(
      x_ref[...], w_ref[...], (((1,), (1,)), ((), ())),
      preferred_element_type=jnp.float32)


_proj = pl.pallas_call(
    _proj_body,
    grid=(NB1,),
    in_specs=[
        pl.BlockSpec((BLK, D_FEAT), lambda i: (i, 0)),
        pl.BlockSpec((HIDDEN, D_FEAT), lambda i: (0, 0)),
    ],
    out_specs=pl.BlockSpec((BLK, HIDDEN), lambda i: (i, 0)),
    out_shape=jax.ShapeDtypeStruct((NPAD, HIDDEN), jnp.float32),
)


def _stage2_body(y_ref, a0_ref, a1_ref, ba_ref, wb_ref, bb_ref, wn_ref, o_ref):
  u = jnp.maximum(y_ref[...] + a0_ref[...] + a1_ref[...] + ba_ref[...], 0.0)
  h = jnp.maximum(
      lax.dot_general(u, wb_ref[...], (((1,), (1,)), ((), ())),
                      preferred_element_type=jnp.float32) + bb_ref[...], 0.0)
  o_ref[...] = lax.dot_general(
      h, wn_ref[...], (((1,), (1,)), ((), ())),
      preferred_element_type=jnp.float32)


_stage2 = pl.pallas_call(
    _stage2_body,
    grid=(NB1,),
    in_specs=[
        pl.BlockSpec((BLK, HIDDEN), lambda i: (i, 0)),
        pl.BlockSpec((BLK, HIDDEN), lambda i: (i, 0)),
        pl.BlockSpec((BLK, HIDDEN), lambda i: (i, 0)),
        pl.BlockSpec((1, HIDDEN), lambda i: (0, 0)),
        pl.BlockSpec((HIDDEN, HIDDEN), lambda i: (0, 0)),
        pl.BlockSpec((1, HIDDEN), lambda i: (0, 0)),
        pl.BlockSpec((HIDDEN, HIDDEN), lambda i: (0, 0)),
    ],
    out_specs=pl.BlockSpec((BLK, HIDDEN), lambda i: (i, 0)),
    out_shape=jax.ShapeDtypeStruct((NPAD, HIDDEN), jnp.float32),
)


def _stage3_body(y_ref, a0_ref, a1_ref, ba_ref, wb_ref, bb_ref, batch_ref,
                 wfc_ref, bfc_ref, o_ref, sums_ref, cnts_ref):
  i = pl.program_id(0)

  @pl.when(i == 0)
  def _():
    sums_ref[...] = jnp.zeros_like(sums_ref)
    cnts_ref[...] = jnp.zeros_like(cnts_ref)

  u = jnp.maximum(y_ref[...] + a0_ref[...] + a1_ref[...] + ba_ref[...], 0.0)
  h = jnp.maximum(
      lax.dot_general(u, wb_ref[...], (((1,), (1,)), ((), ())),
                      preferred_element_type=jnp.float32) + bb_ref[...], 0.0)
  bids = batch_ref[0]
  gids = lax.broadcasted_iota(jnp.int32, (N_GRAPHS, BLK3), 0)
  m = (bids == gids).astype(jnp.float32)
  sums_ref[...] += lax.dot_general(
      m, h, (((1,), (0,)), ((), ())), preferred_element_type=jnp.float32)
  cnts_ref[...] += jnp.broadcast_to(
      jnp.sum(m, axis=1, keepdims=True), (N_GRAPHS, HIDDEN))

  @pl.when(i == NB3 - 1)
  def _():
    pooled = sums_ref[...] / jnp.maximum(cnts_ref[...], 1.0)
    o_ref[...] = lax.dot_general(
        pooled, wfc_ref[...], (((1,), (1,)), ((), ())),
        preferred_element_type=jnp.float32) + bfc_ref[...]


_stage3 = pl.pallas_call(
    _stage3_body,
    grid=(NB3,),
    in_specs=[
        pl.BlockSpec((BLK3, HIDDEN), lambda i: (i, 0)),
        pl.BlockSpec((BLK3, HIDDEN), lambda i: (i, 0)),
        pl.BlockSpec((BLK3, HIDDEN), lambda i: (i, 0)),
        pl.BlockSpec((1, HIDDEN), lambda i: (0, 0)),
        pl.BlockSpec((HIDDEN, HIDDEN), lambda i: (0, 0)),
        pl.BlockSpec((1, HIDDEN), lambda i: (0, 0)),
        pl.BlockSpec((1, 1, BLK3), lambda i: (i, 0, 0)),
        pl.BlockSpec((N_CLASSES, HIDDEN), lambda i: (0, 0)),
        pl.BlockSpec((1, N_CLASSES), lambda i: (0, 0)),
    ],
    out_specs=pl.BlockSpec((N_GRAPHS, N_CLASSES), lambda i: (0, 0)),
    out_shape=jax.ShapeDtypeStruct((N_GRAPHS, N_CLASSES), jnp.float32),
    scratch_shapes=[
        pltpu.VMEM((N_GRAPHS, HIDDEN), jnp.float32),
        pltpu.VMEM((N_GRAPHS, HIDDEN), jnp.float32),
    ],
)


@jax.jit
def kernel(x, edge_index, batch, W1a, b1a, W1b, b1b, W2a, b2a, W2b, b2b,
           Wfc, bfc):
  ei = edge_index.astype(jnp.int32)
  epad = EPAD - N_EDGES
  fill = N_NODES + (jnp.arange(epad, dtype=jnp.int32) % (NPAD - N_NODES))
  src = jnp.concatenate([ei[0], fill]).reshape(NW, CHUNKS, CH)
  dst = jnp.concatenate([ei[1], fill]).reshape(NW, CHUNKS, CH)
  xpad = jnp.pad(x, ((0, NPAD - N_NODES), (0, 0)))
  bpad = jnp.pad(batch.astype(jnp.int32), (0, NPAD - N_NODES),
                 constant_values=N_GRAPHS).reshape(NB3, 1, BLK3)
  zeros = jnp.zeros((NPAD, HIDDEN), jnp.float32)

  y1 = _proj(xpad, W1a)
  a1 = _sc_aggregate(y1, src, dst, zeros)
  y2 = _stage2(y1, a1[0], a1[1], b1a.reshape(1, HIDDEN), W1b,
               b1b.reshape(1, HIDDEN), W2a)
  a2 = _sc_aggregate(y2, src, dst, zeros)
  out = _stage3(y2, a2[0], a2[1], b2a.reshape(1, HIDDEN), W2b,
                b2b.reshape(1, HIDDEN), bpad, Wfc,
                bfc.reshape(1, N_CLASSES))
  return out


# speedup vs baseline: 15.0296x; 1.1202x over previous
import functools

import jax
import jax.numpy as jnp
from jax import lax
from jax.experimental import pallas as pl
from jax.experimental.pallas import tpu as pltpu
from jax.experimental.pallas import tpu_sc as plsc

N_NODES = 10000
N_EDGES = 320000
D_FEAT = 128
HIDDEN = 64
N_CLASSES = 2
N_GRAPHS = 16

NC = 2
NS = 16
NW = NC * NS

NPAD = 10240
CH = 128
EPW = 10240
EPAD = EPW * NW
CHUNKS = EPW // CH
RPW = NPAD // NS

_SC_MESH = plsc.VectorSubcoreMesh(
    core_axis_name="c", subcore_axis_name="s", num_cores=NC, num_subcores=NS)


NBUF = 8
GROUPS = CHUNKS // NBUF


def _agg_body(y_ref, src_ref, dst_ref, zero_ref, out_ref, *scratch):
  sidx, didx = scratch[0], scratch[1]
  rows = scratch[2:2 + NBUF]
  gsem = scratch[2 + NBUF:2 + 2 * NBUF]
  ssem = scratch[2 + 2 * NBUF:2 + 3 * NBUF]
  acc = scratch[2 + 3 * NBUF]
  c = lax.axis_index("c")
  s = lax.axis_index("s")
  w = c * NS + s
  rbase = s * RPW
  pltpu.sync_copy(src_ref.at[w], sidx)
  pltpu.sync_copy(dst_ref.at[w], didx)
  pltpu.sync_copy(zero_ref.at[pl.ds(rbase, RPW)], acc.at[pl.ds(rbase, RPW)])
  plsc.subcore_barrier()

  for b in range(NBUF):
    pltpu.async_copy(y_ref.at[sidx.at[b]], rows[b], gsem[b])

  def group(j, carry):
    base = j * NBUF
    for b in range(NBUF):
      pltpu.make_async_copy(y_ref.at[sidx.at[base + b]], rows[b],
                            gsem[b]).wait()
      pltpu.async_copy(rows[b], acc.at[didx.at[base + b]], ssem[b], add=True)
    for b in range(NBUF):
      nxt = jnp.minimum(base + NBUF + b, CHUNKS - 1)
      pltpu.make_async_copy(rows[b], acc.at[didx.at[nxt]], ssem[b]).wait()

      @pl.when(j < GROUPS - 1)
      def _():
        pltpu.async_copy(y_ref.at[sidx.at[nxt]], rows[b], gsem[b])
    return carry

  lax.fori_loop(0, GROUPS, group, 0)
  plsc.subcore_barrier()
  pltpu.sync_copy(acc.at[pl.ds(rbase, RPW)], out_ref.at[c, pl.ds(rbase, RPW)])


_sc_aggregate = pl.kernel(
    _agg_body,
    out_type=jax.ShapeDtypeStruct((NC, NPAD, HIDDEN), jnp.float32),
    mesh=_SC_MESH,
    scratch_types=[
        pltpu.VMEM((CHUNKS, CH), jnp.int32),
        pltpu.VMEM((CHUNKS, CH), jnp.int32),
    ] + [pltpu.VMEM((CH, HIDDEN), jnp.float32) for _ in range(NBUF)]
    + [pltpu.SemaphoreType.DMA for _ in range(2 * NBUF)]
    + [pltpu.VMEM_SHARED((NPAD, HIDDEN), jnp.float32)],
    compiler_params=pltpu.CompilerParams(use_tc_tiling_on_sc=False),
)


BLK = 2048
NB1 = NPAD // BLK

BLK3 = 1024
NB3 = NPAD // BLK3


def _proj_body(x_ref, w_ref, o_ref):
  o_ref[...] = lax.dot_general(
      x_ref[...], w_ref[...], (((1,), (1,)), ((), ())),
      preferred_element_type=jnp.float32)


_proj = pl.pallas_call(
    _proj_body,
    grid=(NB1,),
    in_specs=[
        pl.BlockSpec((BLK, D_FEAT), lambda i: (i, 0)),
        pl.BlockSpec((HIDDEN, D_FEAT), lambda i: (0, 0)),
    ],
    out_specs=pl.BlockSpec((BLK, HIDDEN), lambda i: (i, 0)),
    out_shape=jax.ShapeDtypeStruct((NPAD, HIDDEN), jnp.float32),
)


def _stage2_body(y_ref, a_ref, ba_ref, wb_ref, bb_ref, wn_ref, o_ref):
  u = jnp.maximum(y_ref[...] + a_ref[0] + a_ref[1] + ba_ref[...], 0.0)
  h = jnp.maximum(
      lax.dot_general(u, wb_ref[...], (((1,), (1,)), ((), ())),
                      preferred_element_type=jnp.float32) + bb_ref[...], 0.0)
  o_ref[...] = lax.dot_general(
      h, wn_ref[...], (((1,), (1,)), ((), ())),
      preferred_element_type=jnp.float32)


_stage2 = pl.pallas_call(
    _stage2_body,
    grid=(NB1,),
    in_specs=[
        pl.BlockSpec((BLK, HIDDEN), lambda i: (i, 0)),
        pl.BlockSpec((NC, BLK, HIDDEN), lambda i: (0, i, 0)),
        pl.BlockSpec((1, HIDDEN), lambda i: (0, 0)),
        pl.BlockSpec((HIDDEN, HIDDEN), lambda i: (0, 0)),
        pl.BlockSpec((1, HIDDEN), lambda i: (0, 0)),
        pl.BlockSpec((HIDDEN, HIDDEN), lambda i: (0, 0)),
    ],
    out_specs=pl.BlockSpec((BLK, HIDDEN), lambda i: (i, 0)),
    out_shape=jax.ShapeDtypeStruct((NPAD, HIDDEN), jnp.float32),
)


def _stage3_body(y_ref, a_ref, ba_ref, wb_ref, bb_ref, batch_ref,
                 wfc_ref, bfc_ref, o_ref, sums_ref, cnts_ref):
  i = pl.program_id(0)

  @pl.when(i == 0)
  def _():
    sums_ref[...] = jnp.zeros_like(sums_ref)
    cnts_ref[...] = jnp.zeros_like(cnts_ref)

  u = jnp.maximum(y_ref[...] + a_ref[0] + a_ref[1] + ba_ref[...], 0.0)
  h = jnp.maximum(
      lax.dot_general(u, wb_ref[...], (((1,), (1,)), ((), ())),
                      preferred_element_type=jnp.float32) + bb_ref[...], 0.0)
  bids = batch_ref[0]
  gids = lax.broadcasted_iota(jnp.int32, (N_GRAPHS, BLK3), 0)
  m = (bids == gids).astype(jnp.float32)
  sums_ref[...] += lax.dot_general(
      m, h, (((1,), (0,)), ((), ())), preferred_element_type=jnp.float32)
  cnts_ref[...] += jnp.broadcast_to(
      jnp.sum(m, axis=1, keepdims=True), (N_GRAPHS, HIDDEN))

  @pl.when(i == NB3 - 1)
  def _():
    pooled = sums_ref[...] / jnp.maximum(cnts_ref[...], 1.0)
    o_ref[...] = lax.dot_general(
        pooled, wfc_ref[...], (((1,), (1,)), ((), ())),
        preferred_element_type=jnp.float32) + bfc_ref[...]


_stage3 = pl.pallas_call(
    _stage3_body,
    grid=(NB3,),
    in_specs=[
        pl.BlockSpec((BLK3, HIDDEN), lambda i: (i, 0)),
        pl.BlockSpec((NC, BLK3, HIDDEN), lambda i: (0, i, 0)),
        pl.BlockSpec((1, HIDDEN), lambda i: (0, 0)),
        pl.BlockSpec((HIDDEN, HIDDEN), lambda i: (0, 0)),
        pl.BlockSpec((1, HIDDEN), lambda i: (0, 0)),
        pl.BlockSpec((1, 1, BLK3), lambda i: (i, 0, 0)),
        pl.BlockSpec((N_CLASSES, HIDDEN), lambda i: (0, 0)),
        pl.BlockSpec((1, N_CLASSES), lambda i: (0, 0)),
    ],
    out_specs=pl.BlockSpec((N_GRAPHS, N_CLASSES), lambda i: (0, 0)),
    out_shape=jax.ShapeDtypeStruct((N_GRAPHS, N_CLASSES), jnp.float32),
    scratch_shapes=[
        pltpu.VMEM((N_GRAPHS, HIDDEN), jnp.float32),
        pltpu.VMEM((N_GRAPHS, HIDDEN), jnp.float32),
    ],
)


@jax.jit
def kernel(x, edge_index, batch, W1a, b1a, W1b, b1b, W2a, b2a, W2b, b2b,
           Wfc, bfc):
  ei = edge_index.astype(jnp.int32)
  epad = EPAD - N_EDGES
  fill = N_NODES + (jnp.arange(epad, dtype=jnp.int32) % (NPAD - N_NODES))
  src = jnp.concatenate([ei[0], fill]).reshape(NW, CHUNKS, CH)
  dst = jnp.concatenate([ei[1], fill]).reshape(NW, CHUNKS, CH)
  xpad = jnp.pad(x, ((0, NPAD - N_NODES), (0, 0)))
  bpad = jnp.pad(batch.astype(jnp.int32), (0, NPAD - N_NODES),
                 constant_values=N_GRAPHS).reshape(NB3, 1, BLK3)
  zeros = jnp.zeros((NPAD, HIDDEN), jnp.float32)

  y1 = _proj(xpad, W1a)
  a1 = _sc_aggregate(y1, src, dst, zeros)
  y2 = _stage2(y1, a1, b1a.reshape(1, HIDDEN), W1b,
               b1b.reshape(1, HIDDEN), W2a)
  a2 = _sc_aggregate(y2, src, dst, zeros)
  out = _stage3(y2, a2, b2a.reshape(1, HIDDEN), W2b,
                b2b.reshape(1, HIDDEN), bpad, Wfc,
                bfc.reshape(1, N_CLASSES))
  return out
